# baseline (speedup 1.0000x reference)
import functools

import jax
import jax.numpy as jnp
from jax import lax
from jax.experimental import pallas as pl
from jax.experimental.pallas import tpu as pltpu
from jax.experimental.pallas import tpu_sc as plsc

_N = 10000
_E = 320000
_D = 128
_H = 128
_C = 20
_TEMP = 10.0

_HE = 144
_HH = _HE // 2
_NC = 2
_NS = 16
_CHUNK = 128
_EPW = _E // _NS
_NCHUNK = _EPW // _CHUNK
_ETAIL = _EPW - _NCHUNK * _CHUNK
_RPS = 632
_AROWS = _NS * _RPS
_TRS = 626
_TROWS = _NS * _TRS

_BN = 2000


def _pre_body(x_ref, win_ref, bin_ref, wr_ref, bl_ref, fe_ref, base_ref):
    xb = x_ref[...]
    feat = jnp.dot(xb, win_ref[...], preferred_element_type=jnp.float32)
    feat = jnp.maximum(feat + bin_ref[...], 0.0)
    fe_ref[0] = feat[:, :_HH]
    col = lax.broadcasted_iota(jnp.int32, (_BN, _HE - _H), 1)
    tail = jnp.where(col == 0, 1.0, 0.0).astype(jnp.float32)
    fe_ref[1] = jnp.concatenate([feat[:, _HH:], tail], axis=1)
    base = jnp.dot(feat, wr_ref[...], preferred_element_type=jnp.float32)
    base_ref[...] = base + bl_ref[...]


def _pre(x, w_in, b_in, w_r, b_l):
    grid = _N // _BN
    return pl.pallas_call(
        _pre_body,
        grid=(grid,),
        in_specs=[
            pl.BlockSpec((_BN, _D), lambda i: (i, 0)),
            pl.BlockSpec((_D, _H), lambda i: (0, 0)),
            pl.BlockSpec((1, _H), lambda i: (0, 0)),
            pl.BlockSpec((_H, _H), lambda i: (0, 0)),
            pl.BlockSpec((1, _H), lambda i: (0, 0)),
        ],
        out_specs=[
            pl.BlockSpec((_NC, _BN, _HH), lambda i: (0, i, 0)),
            pl.BlockSpec((_BN, _H), lambda i: (i, 0)),
        ],
        out_shape=[
            jax.ShapeDtypeStruct((_NC, _TROWS, _HH), jnp.float32),
            jax.ShapeDtypeStruct((_N, _H), jnp.float32),
        ],
    )(x, w_in, b_in, w_r, b_l)


def _sc_body(feat_hbm, edge_hbm, zeros_hbm, out_hbm,
             iring, tsrc_v, tdst_v, rbuf, table_sh, acc_sh,
             isem0, isem1, isem2, isem3, gsem0, gsem1, tsem):
    c = lax.axis_index("c")
    s = lax.axis_index("s")

    isems = [isem0, isem1, isem2, isem3]
    gsems = [gsem0, gsem1]

    def _icopies(chunk_j, slot):
        off = s * _EPW + chunk_j * _CHUNK
        return (
            pltpu.make_async_copy(edge_hbm.at[0, pl.ds(off, _CHUNK)],
                                  iring.at[slot, 0], isems[slot]),
            pltpu.make_async_copy(edge_hbm.at[1, pl.ds(off, _CHUNK)],
                                  iring.at[slot, 1], isems[slot]),
        )

    def i_start(chunk_j, slot):
        for cp in _icopies(chunk_j, slot):
            cp.start()

    def i_wait(chunk_j, slot):
        for cp in _icopies(chunk_j, slot):
            cp.wait()

    def g_start(slot, rb):
        pltpu.make_async_copy(table_sh.at[iring.at[slot, 0]], rbuf.at[rb],
                              gsems[rb]).start()

    def g_wait(slot, rb):
        pltpu.make_async_copy(table_sh.at[iring.at[slot, 0]], rbuf.at[rb],
                              gsems[rb]).wait()

    def scat(slot, rb):
        pltpu.sync_copy(rbuf.at[rb], acc_sh.at[iring.at[slot, 1]], add=True)

    toff = s * _EPW + _NCHUNK * _CHUNK
    tcp0 = pltpu.make_async_copy(edge_hbm.at[0, pl.ds(toff, _ETAIL)],
                                 tsrc_v, tsem)
    tcp0.start()
    tcp1 = pltpu.make_async_copy(edge_hbm.at[1, pl.ds(toff, _ETAIL)],
                                 tdst_v, tsem)
    tcp1.start()

    for k in range(4):
        i_start(k, k)

    pltpu.sync_copy(feat_hbm.at[c, pl.ds(s * _TRS, _TRS)],
                    table_sh.at[pl.ds(s * _TRS, _TRS)])
    pltpu.sync_copy(zeros_hbm, acc_sh.at[pl.ds(s * _RPS, _RPS)])
    plsc.subcore_barrier()

    i_wait(0, 0)
    g_start(0, 0)

    def quad(g, _):
        base_c = 4 * g
        for k in range(4):
            nslot = (k + 1) % 4
            i_wait(base_c + k + 1, nslot)
            g_start(nslot, (k + 1) % 2)
            g_wait(k, k % 2)
            scat(k, k % 2)
            i_start(base_c + 4 + k, k)
        return 0
    lax.fori_loop(0, (_NCHUNK - 4) // 4, quad, 0)

    nc = _NCHUNK
    i_wait(nc - 3, 1)
    g_start(1, 1)
    g_wait(0, 0)
    scat(0, 0)
    i_wait(nc - 2, 2)
    g_start(2, 0)
    g_wait(1, 1)
    scat(1, 1)
    i_wait(nc - 1, 3)
    g_start(3, 1)
    g_wait(2, 0)
    scat(2, 0)
    g_wait(3, 1)
    scat(3, 1)
    tcp0.wait()
    tcp1.wait()
    tbuf = rbuf.at[0].at[pl.ds(0, _ETAIL)]
    pltpu.async_copy(table_sh.at[tsrc_v], tbuf, gsem0).wait()
    pltpu.sync_copy(tbuf, acc_sh.at[tdst_v], add=True)
    plsc.subcore_barrier()

    base = s * _RPS
    pltpu.sync_copy(acc_sh.at[pl.ds(base, _RPS)],
                    out_hbm.at[c, pl.ds(base, _RPS)])


_sc_agg = functools.partial(
    pl.kernel,
    out_type=jax.ShapeDtypeStruct((_NC, _AROWS, _HH), jnp.float32),
    mesh=plsc.VectorSubcoreMesh(core_axis_name="c", subcore_axis_name="s"),
    compiler_params=pltpu.CompilerParams(use_tc_tiling_on_sc=False),
    scratch_types=[
        pltpu.VMEM((4, 2, _CHUNK), jnp.int32),
        pltpu.VMEM((_ETAIL,), jnp.int32),
        pltpu.VMEM((_ETAIL,), jnp.int32),
        pltpu.VMEM((2, _CHUNK, _HH), jnp.float32),
        pltpu.VMEM_SHARED((_TROWS, _HH), jnp.float32),
        pltpu.VMEM_SHARED((_AROWS, _HH), jnp.float32),
        pltpu.SemaphoreType.DMA,
        pltpu.SemaphoreType.DMA,
        pltpu.SemaphoreType.DMA,
        pltpu.SemaphoreType.DMA,
        pltpu.SemaphoreType.DMA,
        pltpu.SemaphoreType.DMA,
        pltpu.SemaphoreType.DMA,
    ],
)(_sc_body)


def _post_body(p0_ref, p1_ref, base_ref, wl_ref, wcls_ref, out_ref, of_ref):
    acc = jnp.concatenate([p0_ref[0], p1_ref[0, :, :_H - _HH]], axis=1)
    cnt = p1_ref[0, :, _H - _HH:_H - _HH + 1]
    mean = acc / jnp.maximum(cnt, 1.0)
    of = jnp.dot(mean, wl_ref[...], preferred_element_type=jnp.float32)
    of = of + base_ref[...]
    of_ref[...] = of
    nrm = jnp.sqrt(jnp.sum(of * of, axis=1, keepdims=True))
    xn = of / jnp.maximum(nrm, 1e-12)
    wc = wcls_ref[...]
    wnrm = jnp.sqrt(jnp.sum(wc * wc, axis=0, keepdims=True))
    wn = wc / jnp.maximum(wnrm, 1e-12)
    out_ref[...] = _TEMP * jnp.dot(xn, wn, preferred_element_type=jnp.float32)


def _post(parts, base, w_l, w_cls):
    grid = _N // _BN
    return pl.pallas_call(
        _post_body,
        grid=(grid,),
        in_specs=[
            pl.BlockSpec((1, _BN, _HH), lambda i: (0, i, 0)),
            pl.BlockSpec((1, _BN, _HH), lambda i: (1, i, 0)),
            pl.BlockSpec((_BN, _H), lambda i: (i, 0)),
            pl.BlockSpec((_H, _H), lambda i: (0, 0)),
            pl.BlockSpec((_H, _C), lambda i: (0, 0)),
        ],
        out_specs=[
            pl.BlockSpec((_BN, _C), lambda i: (i, 0)),
            pl.BlockSpec((_BN, _H), lambda i: (i, 0)),
        ],
        out_shape=[
            jax.ShapeDtypeStruct((_N, _C), jnp.float32),
            jax.ShapeDtypeStruct((_N, _H), jnp.float32),
        ],
    )(parts, parts, base, w_l, w_cls)


def kernel(x, edge_index, W_in, b_in, W_l, b_l, W_r, W_cls):
    feat_split, base = _pre(x, W_in, b_in.reshape(1, _H),
                            W_r, b_l.reshape(1, _H))
    zk = jnp.zeros((_RPS, _HH), jnp.float32)
    parts = _sc_agg(feat_split, edge_index, zk)
    out, out_feat = _post(parts, base, W_l, W_cls)
    return (out, out_feat)


# speedup vs baseline: 1.2158x; 1.2158x over previous
import functools

import jax
import jax.numpy as jnp
from jax import lax
from jax.experimental import pallas as pl
from jax.experimental.pallas import tpu as pltpu
from jax.experimental.pallas import tpu_sc as plsc

_N = 10000
_E = 320000
_D = 128
_H = 128
_C = 20
_TEMP = 10.0

_HE = 144
_NC = 2
_NS = 16
_NW = _NC * _NS
_CHUNK = 128
_EPW = _E // _NW
_NCHUNK = _EPW // _CHUNK
_ETAIL = _EPW - _NCHUNK * _CHUNK
_RPS = 632
_AROWS = _NS * _RPS

_BN = 2000


def _pre_body(x_ref, win_ref, bin_ref, wr_ref, bl_ref, fe_ref, base_ref):
    xb = x_ref[...]
    feat = jnp.dot(xb, win_ref[...], preferred_element_type=jnp.float32)
    feat = jnp.maximum(feat + bin_ref[...], 0.0)
    col = lax.broadcasted_iota(jnp.int32, (_BN, _HE - _H), 1)
    tail = jnp.where(col == 0, 1.0, 0.0).astype(jnp.float32)
    fe_ref[...] = jnp.concatenate([feat, tail], axis=1)
    base = jnp.dot(feat, wr_ref[...], preferred_element_type=jnp.float32)
    base_ref[...] = base + bl_ref[...]


def _pre(x, w_in, b_in, w_r, b_l):
    grid = _N // _BN
    return pl.pallas_call(
        _pre_body,
        grid=(grid,),
        in_specs=[
            pl.BlockSpec((_BN, _D), lambda i: (i, 0)),
            pl.BlockSpec((_D, _H), lambda i: (0, 0)),
            pl.BlockSpec((1, _H), lambda i: (0, 0)),
            pl.BlockSpec((_H, _H), lambda i: (0, 0)),
            pl.BlockSpec((1, _H), lambda i: (0, 0)),
        ],
        out_specs=[
            pl.BlockSpec((_BN, _HE), lambda i: (i, 0)),
            pl.BlockSpec((_BN, _H), lambda i: (i, 0)),
        ],
        out_shape=[
            jax.ShapeDtypeStruct((_N, _HE), jnp.float32),
            jax.ShapeDtypeStruct((_N, _H), jnp.float32),
        ],
    )(x, w_in, b_in, w_r, b_l)


def _sc_body(feat_hbm, edge_hbm, zeros_hbm, out_hbm,
             iring, tsrc_v, tdst_v, rbuf, acc_sh,
             isem0, isem1, isem2, isem3, gsem0, gsem1, tsem):
    c = lax.axis_index("c")
    s = lax.axis_index("s")
    w = s * _NC + c

    isems = [isem0, isem1, isem2, isem3]
    gsems = [gsem0, gsem1]

    def _icopies(chunk_j, slot):
        j = jnp.minimum(chunk_j, _NCHUNK - 1)
        off = w * _EPW + j * _CHUNK
        return (
            pltpu.make_async_copy(edge_hbm.at[0, pl.ds(off, _CHUNK)],
                                  iring.at[slot, 0], isems[slot]),
            pltpu.make_async_copy(edge_hbm.at[1, pl.ds(off, _CHUNK)],
                                  iring.at[slot, 1], isems[slot]),
        )

    def i_start(chunk_j, slot):
        for cp in _icopies(chunk_j, slot):
            cp.start()

    def i_wait(chunk_j, slot):
        for cp in _icopies(chunk_j, slot):
            cp.wait()

    def g_start(slot, rb):
        pltpu.make_async_copy(feat_hbm.at[iring.at[slot, 0]], rbuf.at[rb],
                              gsems[rb]).start()

    def g_wait(slot, rb):
        pltpu.make_async_copy(feat_hbm.at[iring.at[slot, 0]], rbuf.at[rb],
                              gsems[rb]).wait()

    def scat(slot, rb):
        pltpu.sync_copy(rbuf.at[rb], acc_sh.at[iring.at[slot, 1]], add=True)

    toff = w * _EPW + _NCHUNK * _CHUNK
    tcp0 = pltpu.make_async_copy(edge_hbm.at[0, pl.ds(toff, _ETAIL)],
                                 tsrc_v, tsem)
    tcp0.start()
    tcp1 = pltpu.make_async_copy(edge_hbm.at[1, pl.ds(toff, _ETAIL)],
                                 tdst_v, tsem)
    tcp1.start()

    for k in range(4):
        i_start(k, k)
    i_wait(0, 0)
    g_start(0, 0)

    pltpu.sync_copy(zeros_hbm, acc_sh.at[pl.ds(s * _RPS, _RPS)])
    plsc.subcore_barrier()


    def quad(g, _):
        base_c = 4 * g
        for k in range(4):
            nslot = (k + 1) % 4
            i_wait(base_c + k + 1, nslot)
            g_start(nslot, (k + 1) % 2)
            g_wait(k, k % 2)
            scat(k, k % 2)
            i_start(base_c + 4 + k, k)
        return 0
    lax.fori_loop(0, (_NCHUNK - 2) // 4, quad, 0)

    i_wait(_NCHUNK - 1, 1)
    g_start(1, 1)
    g_wait(0, 0)
    scat(0, 0)
    g_wait(1, 1)
    scat(1, 1)
    tcp0.wait()
    tcp1.wait()
    tbuf = rbuf.at[0].at[pl.ds(0, _ETAIL)]
    pltpu.async_copy(feat_hbm.at[tsrc_v], tbuf, gsem0).wait()
    pltpu.sync_copy(tbuf, acc_sh.at[tdst_v], add=True)
    i_wait(_NCHUNK, 2)
    i_wait(_NCHUNK + 1, 3)
    plsc.subcore_barrier()

    base = s * _RPS
    pltpu.sync_copy(acc_sh.at[pl.ds(base, _RPS)],
                    out_hbm.at[c, pl.ds(base, _RPS)])


_sc_agg = functools.partial(
    pl.kernel,
    out_type=jax.ShapeDtypeStruct((_NC, _AROWS, _HE), jnp.float32),
    mesh=plsc.VectorSubcoreMesh(core_axis_name="c", subcore_axis_name="s"),
    compiler_params=pltpu.CompilerParams(use_tc_tiling_on_sc=False),
    scratch_types=[
        pltpu.VMEM((4, 2, _CHUNK), jnp.int32),
        pltpu.VMEM((_ETAIL,), jnp.int32),
        pltpu.VMEM((_ETAIL,), jnp.int32),
        pltpu.VMEM((2, _CHUNK, _HE), jnp.float32),
        pltpu.VMEM_SHARED((_AROWS, _HE), jnp.float32),
        pltpu.SemaphoreType.DMA,
        pltpu.SemaphoreType.DMA,
        pltpu.SemaphoreType.DMA,
        pltpu.SemaphoreType.DMA,
        pltpu.SemaphoreType.DMA,
        pltpu.SemaphoreType.DMA,
        pltpu.SemaphoreType.DMA,
    ],
)(_sc_body)


def _post_body(p0_ref, p1_ref, base_ref, wl_ref, wcls_ref, out_ref, of_ref):
    acc = p0_ref[0, :, :_H] + p1_ref[0, :, :_H]
    cnt = p0_ref[0, :, _H:_H + 1] + p1_ref[0, :, _H:_H + 1]
    mean = acc / jnp.maximum(cnt, 1.0)
    of = jnp.dot(mean, wl_ref[...], preferred_element_type=jnp.float32)
    of = of + base_ref[...]
    of_ref[...] = of
    nrm = jnp.sqrt(jnp.sum(of * of, axis=1, keepdims=True))
    xn = of / jnp.maximum(nrm, 1e-12)
    wc = wcls_ref[...]
    wnrm = jnp.sqrt(jnp.sum(wc * wc, axis=0, keepdims=True))
    wn = wc / jnp.maximum(wnrm, 1e-12)
    out_ref[...] = _TEMP * jnp.dot(xn, wn, preferred_element_type=jnp.float32)


def _post(parts, base, w_l, w_cls):
    grid = _N // _BN
    return pl.pallas_call(
        _post_body,
        grid=(grid,),
        in_specs=[
            pl.BlockSpec((1, _BN, _HE), lambda i: (0, i, 0)),
            pl.BlockSpec((1, _BN, _HE), lambda i: (1, i, 0)),
            pl.BlockSpec((_BN, _H), lambda i: (i, 0)),
            pl.BlockSpec((_H, _H), lambda i: (0, 0)),
            pl.BlockSpec((_H, _C), lambda i: (0, 0)),
        ],
        out_specs=[
            pl.BlockSpec((_BN, _C), lambda i: (i, 0)),
            pl.BlockSpec((_BN, _H), lambda i: (i, 0)),
        ],
        out_shape=[
            jax.ShapeDtypeStruct((_N, _C), jnp.float32),
            jax.ShapeDtypeStruct((_N, _H), jnp.float32),
        ],
    )(parts, parts, base, w_l, w_cls)


def kernel(x, edge_index, W_in, b_in, W_l, b_l, W_r, W_cls):
    feat_ext, base = _pre(x, W_in, b_in.reshape(1, _H),
                          W_r, b_l.reshape(1, _H))
    zk = jnp.zeros((_RPS, _HE), jnp.float32)
    parts = _sc_agg(feat_ext, edge_index, zk)
    out, out_feat = _post(parts, base, W_l, W_cls)
    return (out, out_feat)


# speedup vs baseline: 1.2488x; 1.0271x over previous
import functools

import jax
import jax.numpy as jnp
from jax import lax
from jax.experimental import pallas as pl
from jax.experimental.pallas import tpu as pltpu
from jax.experimental.pallas import tpu_sc as plsc

_N = 10000
_E = 320000
_D = 128
_H = 128
_C = 20
_TEMP = 10.0

_HE = 144
_NC = 2
_NS = 16
_NW = _NC * _NS
_CHUNK = 128
_EPW = _E // _NW
_NCHUNK = _EPW // _CHUNK
_ETAIL = _EPW - _NCHUNK * _CHUNK
_RPS = 632
_AROWS = _NS * _RPS

_BN = 2000


def _pre_body(x_ref, win_ref, bin_ref, wr_ref, bl_ref, fe_ref, base_ref):
    xb = x_ref[...]
    feat = jnp.dot(xb, win_ref[...], preferred_element_type=jnp.float32)
    feat = jnp.maximum(feat + bin_ref[...], 0.0)
    col = lax.broadcasted_iota(jnp.int32, (_BN, _HE - _H), 1)
    tail = jnp.where(col == 0, 1.0, 0.0).astype(jnp.float32)
    fe_ref[...] = jnp.concatenate([feat, tail], axis=1)
    base = jnp.dot(feat, wr_ref[...], preferred_element_type=jnp.float32)
    base_ref[...] = base + bl_ref[...]


def _pre(x, w_in, b_in, w_r, b_l):
    grid = _N // _BN
    return pl.pallas_call(
        _pre_body,
        grid=(grid,),
        in_specs=[
            pl.BlockSpec((_BN, _D), lambda i: (i, 0)),
            pl.BlockSpec((_D, _H), lambda i: (0, 0)),
            pl.BlockSpec((1, _H), lambda i: (0, 0)),
            pl.BlockSpec((_H, _H), lambda i: (0, 0)),
            pl.BlockSpec((1, _H), lambda i: (0, 0)),
        ],
        out_specs=[
            pl.BlockSpec((_BN, _HE), lambda i: (i, 0)),
            pl.BlockSpec((_BN, _H), lambda i: (i, 0)),
        ],
        out_shape=[
            jax.ShapeDtypeStruct((_N, _HE), jnp.float32),
            jax.ShapeDtypeStruct((_N, _H), jnp.float32),
        ],
    )(x, w_in, b_in, w_r, b_l)


def _sc_body(feat_hbm, edge_hbm, out_hbm,
             iring, tsrc_v, tdst_v, rbuf, acc_sh,
             isem0, isem1, isem2, isem3, gsem0, gsem1, tsem):
    c = lax.axis_index("c")
    s = lax.axis_index("s")
    w = s * _NC + c

    isems = [isem0, isem1, isem2, isem3]
    gsems = [gsem0, gsem1]

    def _icopies(chunk_j, slot):
        j = jnp.minimum(chunk_j, _NCHUNK - 1)
        off = w * _EPW + j * _CHUNK
        return (
            pltpu.make_async_copy(edge_hbm.at[0, pl.ds(off, _CHUNK)],
                                  iring.at[slot, 0], isems[slot]),
            pltpu.make_async_copy(edge_hbm.at[1, pl.ds(off, _CHUNK)],
                                  iring.at[slot, 1], isems[slot]),
        )

    def i_start(chunk_j, slot):
        for cp in _icopies(chunk_j, slot):
            cp.start()

    def i_wait(chunk_j, slot):
        for cp in _icopies(chunk_j, slot):
            cp.wait()

    def g_start(slot, rb):
        pltpu.make_async_copy(feat_hbm.at[iring.at[slot, 0]], rbuf.at[rb],
                              gsems[rb]).start()

    def g_wait(slot, rb):
        pltpu.make_async_copy(feat_hbm.at[iring.at[slot, 0]], rbuf.at[rb],
                              gsems[rb]).wait()

    def scat(slot, rb):
        pltpu.sync_copy(rbuf.at[rb], acc_sh.at[iring.at[slot, 1]], add=True)

    toff = w * _EPW + _NCHUNK * _CHUNK
    tcp0 = pltpu.make_async_copy(edge_hbm.at[0, pl.ds(toff, _ETAIL)],
                                 tsrc_v, tsem)
    tcp0.start()
    tcp1 = pltpu.make_async_copy(edge_hbm.at[1, pl.ds(toff, _ETAIL)],
                                 tdst_v, tsem)
    tcp1.start()

    for k in range(4):
        i_start(k, k)

    z = rbuf.at[1]
    def zrow(i, _):
        def zcol(j, _):
            z[i, pl.ds(j * 16, 16)] = jnp.zeros((16,), jnp.float32)
            return 0
        return lax.fori_loop(0, _HE // 16, zcol, 0)
    lax.fori_loop(0, _CHUNK, zrow, 0)

    i_wait(0, 0)
    g_start(0, 0)

    def zcp(t, _):
        pltpu.sync_copy(z, acc_sh.at[pl.ds(s * _RPS + t * _CHUNK, _CHUNK)])
        return 0
    lax.fori_loop(0, _RPS // _CHUNK, zcp, 0)

    ztail = _RPS - (_RPS // _CHUNK) * _CHUNK
    def zcp8(t, _):
        pltpu.sync_copy(
            z.at[pl.ds(0, 8)],
            acc_sh.at[pl.ds(s * _RPS + (_RPS // _CHUNK) * _CHUNK + t * 8, 8)])
        return 0
    lax.fori_loop(0, ztail // 8, zcp8, 0)
    plsc.subcore_barrier()


    def quad(g, _):
        base_c = 4 * g
        for k in range(4):
            nslot = (k + 1) % 4
            i_wait(base_c + k + 1, nslot)
            g_start(nslot, (k + 1) % 2)
            g_wait(k, k % 2)
            scat(k, k % 2)
            i_start(base_c + 4 + k, k)
        return 0
    lax.fori_loop(0, (_NCHUNK - 2) // 4, quad, 0)

    i_wait(_NCHUNK - 1, 1)
    g_start(1, 1)
    g_wait(0, 0)
    scat(0, 0)
    g_wait(1, 1)
    scat(1, 1)
    tcp0.wait()
    tcp1.wait()
    tbuf = rbuf.at[0].at[pl.ds(0, _ETAIL)]
    pltpu.async_copy(feat_hbm.at[tsrc_v], tbuf, gsem0).wait()
    pltpu.sync_copy(tbuf, acc_sh.at[tdst_v], add=True)
    i_wait(_NCHUNK, 2)
    i_wait(_NCHUNK + 1, 3)
    plsc.subcore_barrier()

    base = s * _RPS
    pltpu.sync_copy(acc_sh.at[pl.ds(base, _RPS)],
                    out_hbm.at[c, pl.ds(base, _RPS)])


_sc_agg = functools.partial(
    pl.kernel,
    out_type=jax.ShapeDtypeStruct((_NC, _AROWS, _HE), jnp.float32),
    mesh=plsc.VectorSubcoreMesh(core_axis_name="c", subcore_axis_name="s"),
    compiler_params=pltpu.CompilerParams(use_tc_tiling_on_sc=False),
    scratch_types=[
        pltpu.VMEM((4, 2, _CHUNK), jnp.int32),
        pltpu.VMEM((_ETAIL,), jnp.int32),
        pltpu.VMEM((_ETAIL,), jnp.int32),
        pltpu.VMEM((2, _CHUNK, _HE), jnp.float32),
        pltpu.VMEM_SHARED((_AROWS, _HE), jnp.float32),
        pltpu.SemaphoreType.DMA,
        pltpu.SemaphoreType.DMA,
        pltpu.SemaphoreType.DMA,
        pltpu.SemaphoreType.DMA,
        pltpu.SemaphoreType.DMA,
        pltpu.SemaphoreType.DMA,
        pltpu.SemaphoreType.DMA,
    ],
)(_sc_body)


def _post_body(p0_ref, p1_ref, base_ref, wl_ref, wcls_ref, out_ref, of_ref):
    acc = p0_ref[0, :, :_H] + p1_ref[0, :, :_H]
    cnt = p0_ref[0, :, _H:_H + 1] + p1_ref[0, :, _H:_H + 1]
    mean = acc / jnp.maximum(cnt, 1.0)
    of = jnp.dot(mean, wl_ref[...], preferred_element_type=jnp.float32)
    of = of + base_ref[...]
    of_ref[...] = of
    nrm = jnp.sqrt(jnp.sum(of * of, axis=1, keepdims=True))
    xn = of / jnp.maximum(nrm, 1e-12)
    wc = wcls_ref[...]
    wnrm = jnp.sqrt(jnp.sum(wc * wc, axis=0, keepdims=True))
    wn = wc / jnp.maximum(wnrm, 1e-12)
    out_ref[...] = _TEMP * jnp.dot(xn, wn, preferred_element_type=jnp.float32)


def _post(parts, base, w_l, w_cls):
    grid = _N // _BN
    return pl.pallas_call(
        _post_body,
        grid=(grid,),
        in_specs=[
            pl.BlockSpec((1, _BN, _HE), lambda i: (0, i, 0)),
            pl.BlockSpec((1, _BN, _HE), lambda i: (1, i, 0)),
            pl.BlockSpec((_BN, _H), lambda i: (i, 0)),
            pl.BlockSpec((_H, _H), lambda i: (0, 0)),
            pl.BlockSpec((_H, _C), lambda i: (0, 0)),
        ],
        out_specs=[
            pl.BlockSpec((_BN, _C), lambda i: (i, 0)),
            pl.BlockSpec((_BN, _H), lambda i: (i, 0)),
        ],
        out_shape=[
            jax.ShapeDtypeStruct((_N, _C), jnp.float32),
            jax.ShapeDtypeStruct((_N, _H), jnp.float32),
        ],
    )(parts, parts, base, w_l, w_cls)


def kernel(x, edge_index, W_in, b_in, W_l, b_l, W_r, W_cls):
    feat_ext, base = _pre(x, W_in, b_in.reshape(1, _H),
                          W_r, b_l.reshape(1, _H))
    parts = _sc_agg(feat_ext, edge_index)
    out, out_feat = _post(parts, base, W_l, W_cls)
    return (out, out_feat)
